# baseline (speedup 1.0000x reference)
import functools

import jax
import jax.numpy as jnp
from jax import lax
from jax.experimental import pallas as pl
from jax.experimental.pallas import tpu as pltpu
from jax.experimental.pallas import tpu_sc as plsc

_N = 10000
_NP = 10240
_E = 320000
_EROWS_P = 2560
_RPT = 80
_C0_ROWS = 640
_B = 64
_R = 512
_G = _NP // _R
_MESH = dict(core_axis_name="c", subcore_axis_name="s")


def _sc_aggregate(feat, srcp, dstp, z128):
  ph = 40
  scratch = [
      pltpu.VMEM((ph, 128), jnp.int32),
      pltpu.VMEM((ph, 128), jnp.int32),
      pltpu.VMEM((128, 128), jnp.float32),
      pltpu.VMEM((128, 128), jnp.float32),
      pltpu.VMEM_SHARED((_NP, 128), jnp.float32),
      pltpu.SemaphoreType.DMA,
      pltpu.SemaphoreType.DMA,
  ]

  @functools.partial(
      pl.kernel,
      out_type=jax.ShapeDtypeStruct((2 * _NP, 128), jnp.float32),
      mesh=plsc.VectorSubcoreMesh(**_MESH),
      scratch_types=scratch)
  def run(feat_hbm, srcp_hbm, dstp_hbm, z128_hbm, agg_hbm,
          src_v, dst_v, rows_a, rows_b, agg_sh, sem_a, sem_b):
    c = lax.axis_index("c")
    s = lax.axis_index("s")
    nbase = s * (_NP // 16)

    @pl.loop(0, _NP // 16 // 128)
    def _(k):
      pltpu.sync_copy(z128_hbm, agg_sh.at[pl.ds(nbase + k * 128, 128)])

    plsc.subcore_barrier()

    def pipe(base, nrows):
      for p in range(nrows // ph):
        pltpu.sync_copy(srcp_hbm.at[pl.ds(base + p * ph, ph)], src_v)
        pltpu.sync_copy(dstp_hbm.at[pl.ds(base + p * ph, ph)], dst_v)
        pltpu.async_copy(feat_hbm.at[src_v.at[0]], rows_a, sem_a)
        pltpu.async_copy(feat_hbm.at[src_v.at[1]], rows_b, sem_b)

        @pl.loop(0, ph // 2)
        def _(g):
          pltpu.make_async_copy(z128_hbm, rows_a, sem_a).wait()
          pltpu.sync_copy(rows_a, agg_sh.at[dst_v.at[2 * g]], add=True)

          @pl.when(g < ph // 2 - 1)
          def _():
            pltpu.async_copy(feat_hbm.at[src_v.at[2 * g + 2]], rows_a, sem_a)

          pltpu.make_async_copy(z128_hbm, rows_b, sem_b).wait()
          pltpu.sync_copy(rows_b, agg_sh.at[dst_v.at[2 * g + 1]], add=True)

          @pl.when(g < ph // 2 - 1)
          def _():
            pltpu.async_copy(feat_hbm.at[src_v.at[2 * g + 3]], rows_b, sem_b)

    rpt0 = _C0_ROWS // 16
    rpt1 = (_EROWS_P - _C0_ROWS) // 16

    @pl.when(c == 0)
    def _():
      pipe(s * rpt0, rpt0)

    @pl.when(c == 1)
    def _():
      pipe(_C0_ROWS + s * rpt1, rpt1)

    plsc.subcore_barrier()

    @pl.loop(0, _NP // 16 // 128)
    def _(k):
      off = nbase + k * 128
      pltpu.sync_copy(agg_sh.at[pl.ds(off, 128)],
                      agg_hbm.at[pl.ds(c * _NP + off, 128)])

  return run(feat, srcp, dstp, z128)


def _sc_degree(dstp, z128, ones128):
  scratch = [
      pltpu.VMEM((_RPT, 128), jnp.int32),
      pltpu.VMEM((128, 128), jnp.float32),
      pltpu.VMEM_SHARED((_NP, 128), jnp.float32),
  ]

  @functools.partial(
      pl.kernel,
      out_type=jax.ShapeDtypeStruct((2 * _NP, 128), jnp.float32),
      mesh=plsc.VectorSubcoreMesh(**_MESH),
      scratch_types=scratch)
  def run(dstp_hbm, z128_hbm, ones128_hbm, deg_hbm,
          dst_v, ones_v, deg_sh):
    c = lax.axis_index("c")
    s = lax.axis_index("s")
    ebase = c * (_EROWS_P // 2) + s * _RPT
    nbase = s * (_NP // 16)

    pltpu.sync_copy(ones128_hbm, ones_v)

    @pl.loop(0, _NP // 16 // 128)
    def _(k):
      pltpu.sync_copy(z128_hbm, deg_sh.at[pl.ds(nbase + k * 128, 128)])

    pltpu.sync_copy(dstp_hbm.at[pl.ds(ebase, _RPT)], dst_v)
    plsc.subcore_barrier()

    @pl.loop(0, _RPT)
    def _(r):
      pltpu.sync_copy(ones_v, deg_sh.at[dst_v.at[r]], add=True)

    plsc.subcore_barrier()

    @pl.loop(0, _NP // 16 // 128)
    def _(k):
      off = nbase + k * 128
      pltpu.sync_copy(deg_sh.at[pl.ds(off, 128)],
                      deg_hbm.at[pl.ds(c * _NP + off, 128)])

  return run(dstp, z128, ones128)


def _dot_t(a, w):
  return lax.dot_general(a, w, (((1,), (1,)), ((), ())),
                         preferred_element_type=jnp.float32)


def _tc_layer_body(with_h, x_ref, a0_ref, a1_ref, d0_ref, d1_ref, b3_ref,
                   wrel_ref, wroot_ref, b_ref, *out_refs):
  if with_h:
    h_ref, pool_ref, cnt_ref = out_refs
  else:
    (pool_ref,) = out_refs
  agg = a0_ref[...] + a1_ref[...]
  deg = jnp.maximum((d0_ref[...] + d1_ref[...])[:, 0:1], 1.0)
  h = _dot_t(agg / deg, wrel_ref[...])
  h = h + _dot_t(x_ref[...], wroot_ref[...])
  h = jnp.maximum(h + b_ref[...], 0.0)
  if with_h:
    h_ref[...] = h
  brow = b3_ref[0]
  iot = lax.broadcasted_iota(jnp.int32, (_B, 1), 0)
  m = (brow == iot).astype(jnp.float32)
  psum = lax.dot_general(m, h, (((1,), (0,)), ((), ())),
                         preferred_element_type=jnp.float32)

  @pl.when(pl.program_id(0) == 0)
  def _():
    pool_ref[...] = jnp.zeros_like(pool_ref)
    if with_h:
      cnt_ref[...] = jnp.zeros_like(cnt_ref)

  pool_ref[...] += psum
  if with_h:
    ones8 = jnp.ones((_R, 8), jnp.float32)
    cnt_ref[...] += lax.dot_general(m, ones8, (((1,), (0,)), ((), ())),
                                    preferred_element_type=jnp.float32)


def _tc_layer(xp, agg, deg, batch3, w_rel, w_root, b, with_h):
  row = pl.BlockSpec((_R, 128), lambda i: (i, 0))
  in_specs = [
      row,
      pl.BlockSpec((_R, 128), lambda i: (i, 0)),
      pl.BlockSpec((_R, 128), lambda i: (i + _G, 0)),
      pl.BlockSpec((_R, 128), lambda i: (i, 0)),
      pl.BlockSpec((_R, 128), lambda i: (i + _G, 0)),
      pl.BlockSpec((1, 1, _R), lambda i: (i, 0, 0)),
      pl.BlockSpec((128, 128), lambda i: (0, 0)),
      pl.BlockSpec((128, 128), lambda i: (0, 0)),
      pl.BlockSpec((1, 128), lambda i: (0, 0)),
  ]
  out_shape = [jax.ShapeDtypeStruct((_B, 128), jnp.float32)]
  out_specs = [pl.BlockSpec((_B, 128), lambda i: (0, 0))]
  if with_h:
    out_shape = [jax.ShapeDtypeStruct((_NP, 128), jnp.float32)] + out_shape
    out_specs = [row] + out_specs
    out_shape.append(jax.ShapeDtypeStruct((_B, 8), jnp.float32))
    out_specs.append(pl.BlockSpec((_B, 8), lambda i: (0, 0)))
  return pl.pallas_call(
      functools.partial(_tc_layer_body, with_h),
      grid=(_G,),
      in_specs=in_specs,
      out_specs=out_specs,
      out_shape=out_shape,
  )(xp, agg, agg, deg, deg, batch3, w_rel, w_root, b.reshape(1, 128))


def _tc_head_body(p1_ref, p2_ref, cnt_ref, w1a_ref, w1b_ref, b1_ref,
                  w2_ref, b2_ref, out_ref):
  cnt = jnp.maximum(cnt_ref[...][:, 0:1], 1.0)
  a = _dot_t(p1_ref[...] / cnt, w1a_ref[...])
  a = a + _dot_t(p2_ref[...] / cnt, w1b_ref[...])
  a = jnp.maximum(a + b1_ref[...], 0.0)
  out_ref[...] = _dot_t(a, w2_ref[...]) + b2_ref[...]


def _tc_head(pool1, pool2, cnt, lin1_W, lin1_b, lin2_W, lin2_b):
  return pl.pallas_call(
      _tc_head_body,
      out_shape=jax.ShapeDtypeStruct((_B, 10), jnp.float32),
  )(pool1, pool2, cnt, lin1_W[:, :128], lin1_W[:, 128:],
    lin1_b.reshape(1, 128), lin2_W, lin2_b.reshape(1, 10))


def kernel(x, edge_index, batch, W1_rel, W1_root, b1, W2_rel, W2_root, b2,
           lin1_W, lin1_b, lin2_W, lin2_b):
  src = edge_index[0]
  dst = edge_index[1]
  pad = _EROWS_P * 128 - _E
  srcp = jnp.concatenate([src, jnp.zeros((pad,), jnp.int32)]).reshape(-1, 128)
  trash = _N + (jnp.arange(pad, dtype=jnp.int32) % (_NP - _N))
  dstp = jnp.concatenate([dst, trash]).reshape(-1, 128)
  xp = jnp.zeros((_NP, 128), jnp.float32).at[:_N].set(x)
  batch3 = jnp.concatenate([batch, jnp.full((_NP - _N,), _B, jnp.int32)])
  batch3 = batch3.reshape(_G, 1, _R)
  z128 = jnp.zeros((128, 128), jnp.float32)
  ones128 = jnp.ones((128, 128), jnp.float32)

  deg = _sc_degree(dstp, z128, ones128)
  agg1 = _sc_aggregate(xp, srcp, dstp, z128)
  h1, pool1, cnt = _tc_layer(xp, agg1, deg, batch3, W1_rel, W1_root, b1, True)
  agg2 = _sc_aggregate(h1, srcp, dstp, z128)
  (pool2,) = _tc_layer(h1, agg2, deg, batch3, W2_rel, W2_root, b2, False)
  return _tc_head(pool1, pool2, cnt, lin1_W, lin1_b, lin2_W, lin2_b)


# speedup vs baseline: 1.0965x; 1.0965x over previous
import functools

import jax
import jax.numpy as jnp
from jax import lax
from jax.experimental import pallas as pl
from jax.experimental.pallas import tpu as pltpu
from jax.experimental.pallas import tpu_sc as plsc

_N = 10000
_NP = 10240
_E = 320000
_EROWS_P = 2560
_RPT = 80
_C0_ROWS = 1920
_B = 64
_R = 512
_G = _NP // _R
_MESH = dict(core_axis_name="c", subcore_axis_name="s")


def _sc_aggregate(feat, srcp, dstp, z128):
  ph = 40
  scratch = [
      pltpu.VMEM((ph, 128), jnp.int32),
      pltpu.VMEM((ph, 128), jnp.int32),
      pltpu.VMEM((128, 128), jnp.float32),
      pltpu.VMEM((128, 128), jnp.float32),
      pltpu.VMEM_SHARED((_NP, 128), jnp.float32),
      pltpu.SemaphoreType.DMA,
      pltpu.SemaphoreType.DMA,
  ]

  @functools.partial(
      pl.kernel,
      out_type=jax.ShapeDtypeStruct((2 * _NP, 128), jnp.float32),
      mesh=plsc.VectorSubcoreMesh(**_MESH),
      scratch_types=scratch)
  def run(feat_hbm, srcp_hbm, dstp_hbm, z128_hbm, agg_hbm,
          src_v, dst_v, rows_a, rows_b, agg_sh, sem_a, sem_b):
    c = lax.axis_index("c")
    s = lax.axis_index("s")
    nbase = s * (_NP // 16)

    @pl.loop(0, _NP // 16 // 128)
    def _(k):
      pltpu.sync_copy(z128_hbm, agg_sh.at[pl.ds(nbase + k * 128, 128)])

    plsc.subcore_barrier()

    def pipe(base, nrows):
      for p in range(nrows // ph):
        pltpu.sync_copy(srcp_hbm.at[pl.ds(base + p * ph, ph)], src_v)
        pltpu.sync_copy(dstp_hbm.at[pl.ds(base + p * ph, ph)], dst_v)
        pltpu.async_copy(feat_hbm.at[src_v.at[0]], rows_a, sem_a)
        pltpu.async_copy(feat_hbm.at[src_v.at[1]], rows_b, sem_b)

        @pl.loop(0, ph // 2)
        def _(g):
          pltpu.make_async_copy(z128_hbm, rows_a, sem_a).wait()
          pltpu.sync_copy(rows_a, agg_sh.at[dst_v.at[2 * g]], add=True)

          @pl.when(g < ph // 2 - 1)
          def _():
            pltpu.async_copy(feat_hbm.at[src_v.at[2 * g + 2]], rows_a, sem_a)

          pltpu.make_async_copy(z128_hbm, rows_b, sem_b).wait()
          pltpu.sync_copy(rows_b, agg_sh.at[dst_v.at[2 * g + 1]], add=True)

          @pl.when(g < ph // 2 - 1)
          def _():
            pltpu.async_copy(feat_hbm.at[src_v.at[2 * g + 3]], rows_b, sem_b)

    rpt0 = _C0_ROWS // 16
    rpt1 = (_EROWS_P - _C0_ROWS) // 16

    @pl.when(c == 0)
    def _():
      pipe(s * rpt0, rpt0)

    @pl.when(c == 1)
    def _():
      pipe(_C0_ROWS + s * rpt1, rpt1)

    plsc.subcore_barrier()

    @pl.loop(0, _NP // 16 // 128)
    def _(k):
      off = nbase + k * 128
      pltpu.sync_copy(agg_sh.at[pl.ds(off, 128)],
                      agg_hbm.at[pl.ds(c * _NP + off, 128)])

  return run(feat, srcp, dstp, z128)


def _sc_degree(dstp, z128, ones128):
  scratch = [
      pltpu.VMEM((_RPT, 128), jnp.int32),
      pltpu.VMEM((128, 128), jnp.float32),
      pltpu.VMEM_SHARED((_NP, 128), jnp.float32),
  ]

  @functools.partial(
      pl.kernel,
      out_type=jax.ShapeDtypeStruct((2 * _NP, 128), jnp.float32),
      mesh=plsc.VectorSubcoreMesh(**_MESH),
      scratch_types=scratch)
  def run(dstp_hbm, z128_hbm, ones128_hbm, deg_hbm,
          dst_v, ones_v, deg_sh):
    c = lax.axis_index("c")
    s = lax.axis_index("s")
    ebase = c * (_EROWS_P // 2) + s * _RPT
    nbase = s * (_NP // 16)

    pltpu.sync_copy(ones128_hbm, ones_v)

    @pl.loop(0, _NP // 16 // 128)
    def _(k):
      pltpu.sync_copy(z128_hbm, deg_sh.at[pl.ds(nbase + k * 128, 128)])

    pltpu.sync_copy(dstp_hbm.at[pl.ds(ebase, _RPT)], dst_v)
    plsc.subcore_barrier()

    @pl.loop(0, _RPT)
    def _(r):
      pltpu.sync_copy(ones_v, deg_sh.at[dst_v.at[r]], add=True)

    plsc.subcore_barrier()

    @pl.loop(0, _NP // 16 // 128)
    def _(k):
      off = nbase + k * 128
      pltpu.sync_copy(deg_sh.at[pl.ds(off, 128)],
                      deg_hbm.at[pl.ds(c * _NP + off, 128)])

  return run(dstp, z128, ones128)


def _dot_t(a, w):
  return lax.dot_general(a, w, (((1,), (1,)), ((), ())),
                         preferred_element_type=jnp.float32)


def _tc_layer_body(with_h, x_ref, a0_ref, a1_ref, d0_ref, d1_ref, b3_ref,
                   wrel_ref, wroot_ref, b_ref, *out_refs):
  if with_h:
    h_ref, pool_ref, cnt_ref = out_refs
  else:
    (pool_ref,) = out_refs
  agg = a0_ref[...] + a1_ref[...]
  deg = jnp.maximum((d0_ref[...] + d1_ref[...])[:, 0:1], 1.0)
  h = _dot_t(agg / deg, wrel_ref[...])
  h = h + _dot_t(x_ref[...], wroot_ref[...])
  h = jnp.maximum(h + b_ref[...], 0.0)
  if with_h:
    h_ref[...] = h
  brow = b3_ref[0]
  iot = lax.broadcasted_iota(jnp.int32, (_B, 1), 0)
  m = (brow == iot).astype(jnp.float32)
  psum = lax.dot_general(m, h, (((1,), (0,)), ((), ())),
                         preferred_element_type=jnp.float32)

  @pl.when(pl.program_id(0) == 0)
  def _():
    pool_ref[...] = jnp.zeros_like(pool_ref)
    if with_h:
      cnt_ref[...] = jnp.zeros_like(cnt_ref)

  pool_ref[...] += psum
  if with_h:
    ones8 = jnp.ones((_R, 8), jnp.float32)
    cnt_ref[...] += lax.dot_general(m, ones8, (((1,), (0,)), ((), ())),
                                    preferred_element_type=jnp.float32)


def _tc_layer(xp, agg, deg, batch3, w_rel, w_root, b, with_h):
  row = pl.BlockSpec((_R, 128), lambda i: (i, 0))
  in_specs = [
      row,
      pl.BlockSpec((_R, 128), lambda i: (i, 0)),
      pl.BlockSpec((_R, 128), lambda i: (i + _G, 0)),
      pl.BlockSpec((_R, 128), lambda i: (i, 0)),
      pl.BlockSpec((_R, 128), lambda i: (i + _G, 0)),
      pl.BlockSpec((1, 1, _R), lambda i: (i, 0, 0)),
      pl.BlockSpec((128, 128), lambda i: (0, 0)),
      pl.BlockSpec((128, 128), lambda i: (0, 0)),
      pl.BlockSpec((1, 128), lambda i: (0, 0)),
  ]
  out_shape = [jax.ShapeDtypeStruct((_B, 128), jnp.float32)]
  out_specs = [pl.BlockSpec((_B, 128), lambda i: (0, 0))]
  if with_h:
    out_shape = [jax.ShapeDtypeStruct((_NP, 128), jnp.float32)] + out_shape
    out_specs = [row] + out_specs
    out_shape.append(jax.ShapeDtypeStruct((_B, 8), jnp.float32))
    out_specs.append(pl.BlockSpec((_B, 8), lambda i: (0, 0)))
  return pl.pallas_call(
      functools.partial(_tc_layer_body, with_h),
      grid=(_G,),
      in_specs=in_specs,
      out_specs=out_specs,
      out_shape=out_shape,
  )(xp, agg, agg, deg, deg, batch3, w_rel, w_root, b.reshape(1, 128))


def _tc_head_body(p1_ref, p2_ref, cnt_ref, w1a_ref, w1b_ref, b1_ref,
                  w2_ref, b2_ref, out_ref):
  cnt = jnp.maximum(cnt_ref[...][:, 0:1], 1.0)
  a = _dot_t(p1_ref[...] / cnt, w1a_ref[...])
  a = a + _dot_t(p2_ref[...] / cnt, w1b_ref[...])
  a = jnp.maximum(a + b1_ref[...], 0.0)
  out_ref[...] = _dot_t(a, w2_ref[...]) + b2_ref[...]


def _tc_head(pool1, pool2, cnt, lin1_W, lin1_b, lin2_W, lin2_b):
  return pl.pallas_call(
      _tc_head_body,
      out_shape=jax.ShapeDtypeStruct((_B, 10), jnp.float32),
  )(pool1, pool2, cnt, lin1_W[:, :128], lin1_W[:, 128:],
    lin1_b.reshape(1, 128), lin2_W, lin2_b.reshape(1, 10))


def kernel(x, edge_index, batch, W1_rel, W1_root, b1, W2_rel, W2_root, b2,
           lin1_W, lin1_b, lin2_W, lin2_b):
  src = edge_index[0]
  dst = edge_index[1]
  pad = _EROWS_P * 128 - _E
  srcp = jnp.concatenate([src, jnp.zeros((pad,), jnp.int32)]).reshape(-1, 128)
  trash = _N + (jnp.arange(pad, dtype=jnp.int32) % (_NP - _N))
  dstp = jnp.concatenate([dst, trash]).reshape(-1, 128)
  xp = jnp.zeros((_NP, 128), jnp.float32).at[:_N].set(x)
  batch3 = jnp.concatenate([batch, jnp.full((_NP - _N,), _B, jnp.int32)])
  batch3 = batch3.reshape(_G, 1, _R)
  z128 = jnp.zeros((128, 128), jnp.float32)
  ones128 = jnp.ones((128, 128), jnp.float32)

  deg = _sc_degree(dstp, z128, ones128)
  agg1 = _sc_aggregate(xp, srcp, dstp, z128)
  h1, pool1, cnt = _tc_layer(xp, agg1, deg, batch3, W1_rel, W1_root, b1, True)
  agg2 = _sc_aggregate(h1, srcp, dstp, z128)
  (pool2,) = _tc_layer(h1, agg2, deg, batch3, W2_rel, W2_root, b2, False)
  return _tc_head(pool1, pool2, cnt, lin1_W, lin1_b, lin2_W, lin2_b)


# speedup vs baseline: 1.1517x; 1.0504x over previous
import functools

import jax
import jax.numpy as jnp
from jax import lax
from jax.experimental import pallas as pl
from jax.experimental.pallas import tpu as pltpu
from jax.experimental.pallas import tpu_sc as plsc

_N = 10000
_NP = 10240
_E = 320000
_EROWS_P = 2560
_RPT = 80
_R64 = 5120
_C0_ROWS = 3840
_B = 64
_R = 512
_G = _NP // _R
_MESH = dict(core_axis_name="c", subcore_axis_name="s")


def _sc_aggregate(feat, srcp, dstp, z128):
  ph = 40
  nbuf = 4
  ck = 64
  scratch = [
      pltpu.VMEM((ph, ck), jnp.int32),
      pltpu.VMEM((ph, ck), jnp.int32),
      [pltpu.VMEM((ck, 128), jnp.float32)] * nbuf,
      [pltpu.SemaphoreType.DMA] * nbuf,
      pltpu.VMEM_SHARED((_NP, 128), jnp.float32),
  ]

  @functools.partial(
      pl.kernel,
      out_type=jax.ShapeDtypeStruct((2 * _NP, 128), jnp.float32),
      mesh=plsc.VectorSubcoreMesh(**_MESH),
      scratch_types=scratch)
  def run(feat_hbm, srcp_hbm, dstp_hbm, z128_hbm, agg_hbm,
          src_v, dst_v, rows, sems, agg_sh):
    c = lax.axis_index("c")
    s = lax.axis_index("s")
    nbase = s * (_NP // 16)

    @pl.loop(0, _NP // 16 // 128)
    def _(k):
      pltpu.sync_copy(z128_hbm, agg_sh.at[pl.ds(nbase + k * 128, 128)])

    plsc.subcore_barrier()

    def pipe(base, nrows):
      for p in range(nrows // ph):
        pltpu.sync_copy(srcp_hbm.at[pl.ds(base + p * ph, ph)], src_v)
        pltpu.sync_copy(dstp_hbm.at[pl.ds(base + p * ph, ph)], dst_v)
        for j in range(nbuf):
          pltpu.async_copy(feat_hbm.at[src_v.at[j]], rows[j], sems[j])

        @pl.loop(0, ph // nbuf)
        def _(g):
          for j in range(nbuf):
            pltpu.make_async_copy(z128_hbm.at[pl.ds(0, ck)], rows[j],
                                  sems[j]).wait()
            pltpu.sync_copy(rows[j], agg_sh.at[dst_v.at[nbuf * g + j]],
                            add=True)

            @pl.when(g < ph // nbuf - 1)
            def _():
              pltpu.async_copy(feat_hbm.at[src_v.at[nbuf * (g + 1) + j]],
                               rows[j], sems[j])

    rpt0 = _C0_ROWS // 16
    rpt1 = (_R64 - _C0_ROWS) // 16

    @pl.when(c == 0)
    def _():
      pipe(s * rpt0, rpt0)

    @pl.when(c == 1)
    def _():
      pipe(_C0_ROWS + s * rpt1, rpt1)

    plsc.subcore_barrier()

    @pl.loop(0, _NP // 16 // 128)
    def _(k):
      off = nbase + k * 128
      pltpu.sync_copy(agg_sh.at[pl.ds(off, 128)],
                      agg_hbm.at[pl.ds(c * _NP + off, 128)])

  return run(feat, srcp, dstp, z128)


def _sc_degree(dstp, z128, ones128):
  scratch = [
      pltpu.VMEM((_RPT, 128), jnp.int32),
      pltpu.VMEM((128, 128), jnp.float32),
      pltpu.VMEM_SHARED((_NP, 128), jnp.float32),
  ]

  @functools.partial(
      pl.kernel,
      out_type=jax.ShapeDtypeStruct((2 * _NP, 128), jnp.float32),
      mesh=plsc.VectorSubcoreMesh(**_MESH),
      scratch_types=scratch)
  def run(dstp_hbm, z128_hbm, ones128_hbm, deg_hbm,
          dst_v, ones_v, deg_sh):
    c = lax.axis_index("c")
    s = lax.axis_index("s")
    ebase = c * (_EROWS_P // 2) + s * _RPT
    nbase = s * (_NP // 16)

    pltpu.sync_copy(ones128_hbm, ones_v)

    @pl.loop(0, _NP // 16 // 128)
    def _(k):
      pltpu.sync_copy(z128_hbm, deg_sh.at[pl.ds(nbase + k * 128, 128)])

    pltpu.sync_copy(dstp_hbm.at[pl.ds(ebase, _RPT)], dst_v)
    plsc.subcore_barrier()

    @pl.loop(0, _RPT)
    def _(r):
      pltpu.sync_copy(ones_v, deg_sh.at[dst_v.at[r]], add=True)

    plsc.subcore_barrier()

    @pl.loop(0, _NP // 16 // 128)
    def _(k):
      off = nbase + k * 128
      pltpu.sync_copy(deg_sh.at[pl.ds(off, 128)],
                      deg_hbm.at[pl.ds(c * _NP + off, 128)])

  return run(dstp, z128, ones128)


def _dot_t(a, w):
  return lax.dot_general(a, w, (((1,), (1,)), ((), ())),
                         preferred_element_type=jnp.float32)


def _tc_layer_body(with_h, x_ref, a0_ref, a1_ref, d0_ref, d1_ref, b3_ref,
                   wrel_ref, wroot_ref, b_ref, *out_refs):
  if with_h:
    h_ref, pool_ref, cnt_ref = out_refs
  else:
    (pool_ref,) = out_refs
  agg = a0_ref[...] + a1_ref[...]
  deg = jnp.maximum((d0_ref[...] + d1_ref[...])[:, 0:1], 1.0)
  h = _dot_t(agg / deg, wrel_ref[...])
  h = h + _dot_t(x_ref[...], wroot_ref[...])
  h = jnp.maximum(h + b_ref[...], 0.0)
  if with_h:
    h_ref[...] = h
  brow = b3_ref[0]
  iot = lax.broadcasted_iota(jnp.int32, (_B, 1), 0)
  m = (brow == iot).astype(jnp.float32)
  psum = lax.dot_general(m, h, (((1,), (0,)), ((), ())),
                         preferred_element_type=jnp.float32)

  @pl.when(pl.program_id(0) == 0)
  def _():
    pool_ref[...] = jnp.zeros_like(pool_ref)
    if with_h:
      cnt_ref[...] = jnp.zeros_like(cnt_ref)

  pool_ref[...] += psum
  if with_h:
    ones8 = jnp.ones((_R, 8), jnp.float32)
    cnt_ref[...] += lax.dot_general(m, ones8, (((1,), (0,)), ((), ())),
                                    preferred_element_type=jnp.float32)


def _tc_layer(xp, agg, deg, batch3, w_rel, w_root, b, with_h):
  row = pl.BlockSpec((_R, 128), lambda i: (i, 0))
  in_specs = [
      row,
      pl.BlockSpec((_R, 128), lambda i: (i, 0)),
      pl.BlockSpec((_R, 128), lambda i: (i + _G, 0)),
      pl.BlockSpec((_R, 128), lambda i: (i, 0)),
      pl.BlockSpec((_R, 128), lambda i: (i + _G, 0)),
      pl.BlockSpec((1, 1, _R), lambda i: (i, 0, 0)),
      pl.BlockSpec((128, 128), lambda i: (0, 0)),
      pl.BlockSpec((128, 128), lambda i: (0, 0)),
      pl.BlockSpec((1, 128), lambda i: (0, 0)),
  ]
  out_shape = [jax.ShapeDtypeStruct((_B, 128), jnp.float32)]
  out_specs = [pl.BlockSpec((_B, 128), lambda i: (0, 0))]
  if with_h:
    out_shape = [jax.ShapeDtypeStruct((_NP, 128), jnp.float32)] + out_shape
    out_specs = [row] + out_specs
    out_shape.append(jax.ShapeDtypeStruct((_B, 8), jnp.float32))
    out_specs.append(pl.BlockSpec((_B, 8), lambda i: (0, 0)))
  return pl.pallas_call(
      functools.partial(_tc_layer_body, with_h),
      grid=(_G,),
      in_specs=in_specs,
      out_specs=out_specs,
      out_shape=out_shape,
  )(xp, agg, agg, deg, deg, batch3, w_rel, w_root, b.reshape(1, 128))


def _tc_head_body(p1_ref, p2_ref, cnt_ref, w1a_ref, w1b_ref, b1_ref,
                  w2_ref, b2_ref, out_ref):
  cnt = jnp.maximum(cnt_ref[...][:, 0:1], 1.0)
  a = _dot_t(p1_ref[...] / cnt, w1a_ref[...])
  a = a + _dot_t(p2_ref[...] / cnt, w1b_ref[...])
  a = jnp.maximum(a + b1_ref[...], 0.0)
  out_ref[...] = _dot_t(a, w2_ref[...]) + b2_ref[...]


def _tc_head(pool1, pool2, cnt, lin1_W, lin1_b, lin2_W, lin2_b):
  return pl.pallas_call(
      _tc_head_body,
      out_shape=jax.ShapeDtypeStruct((_B, 10), jnp.float32),
  )(pool1, pool2, cnt, lin1_W[:, :128], lin1_W[:, 128:],
    lin1_b.reshape(1, 128), lin2_W, lin2_b.reshape(1, 10))


def kernel(x, edge_index, batch, W1_rel, W1_root, b1, W2_rel, W2_root, b2,
           lin1_W, lin1_b, lin2_W, lin2_b):
  src = edge_index[0]
  dst = edge_index[1]
  pad = _EROWS_P * 128 - _E
  srcp = jnp.concatenate([src, jnp.zeros((pad,), jnp.int32)]).reshape(-1, 64)
  trash = _N + (jnp.arange(pad, dtype=jnp.int32) % (_NP - _N))
  dstflat = jnp.concatenate([dst, trash])
  dstp = dstflat.reshape(-1, 64)
  dstp128 = dstflat.reshape(-1, 128)
  xp = jnp.zeros((_NP, 128), jnp.float32).at[:_N].set(x)
  batch3 = jnp.concatenate([batch, jnp.full((_NP - _N,), _B, jnp.int32)])
  batch3 = batch3.reshape(_G, 1, _R)
  z128 = jnp.zeros((128, 128), jnp.float32)
  ones128 = jnp.ones((128, 128), jnp.float32)

  deg = _sc_degree(dstp128, z128, ones128)
  agg1 = _sc_aggregate(xp, srcp, dstp, z128)
  h1, pool1, cnt = _tc_layer(xp, agg1, deg, batch3, W1_rel, W1_root, b1, True)
  agg2 = _sc_aggregate(h1, srcp, dstp, z128)
  (pool2,) = _tc_layer(h1, agg2, deg, batch3, W2_rel, W2_root, b2, False)
  return _tc_head(pool1, pool2, cnt, lin1_W, lin1_b, lin2_W, lin2_b)


# speedup vs baseline: 1.1540x; 1.0019x over previous
import functools

import jax
import jax.numpy as jnp
from jax import lax
from jax.experimental import pallas as pl
from jax.experimental.pallas import tpu as pltpu
from jax.experimental.pallas import tpu_sc as plsc

_N = 10000
_NP = 10240
_E = 320000
_EROWS_P = 2560
_RPT = 80
_R64 = 5120
_C0_ROWS = 3840
_B = 64
_R = 512
_G = _NP // _R
_MESH = dict(core_axis_name="c", subcore_axis_name="s")


def _sc_aggregate(feat, srcp, dstp, z128):
  ph = 40
  nbuf = 4
  ck = 64
  scratch = [
      pltpu.VMEM((ph, ck), jnp.int32),
      pltpu.VMEM((ph, ck), jnp.int32),
      [pltpu.VMEM((ck, 128), jnp.float32)] * nbuf,
      [pltpu.SemaphoreType.DMA] * nbuf,
      pltpu.VMEM_SHARED((_NP, 128), jnp.float32),
  ]

  @functools.partial(
      pl.kernel,
      out_type=jax.ShapeDtypeStruct((2 * _NP, 128), jnp.float32),
      mesh=plsc.VectorSubcoreMesh(**_MESH),
      scratch_types=scratch)
  def run(feat_hbm, srcp_hbm, dstp_hbm, z128_hbm, agg_hbm,
          src_v, dst_v, rows, sems, agg_sh):
    c = lax.axis_index("c")
    s = lax.axis_index("s")
    nbase = s * (_NP // 16)

    @pl.loop(0, _NP // 16 // 128)
    def _(k):
      pltpu.sync_copy(z128_hbm, agg_sh.at[pl.ds(nbase + k * 128, 128)])

    plsc.subcore_barrier()

    rpt0 = _C0_ROWS // 16
    rpt1 = (_R64 - _C0_ROWS) // 16
    nph = jnp.where(c == 0, rpt0 // ph, rpt1 // ph)
    base0 = jnp.where(c == 0, s * rpt0, _C0_ROWS + s * rpt1)

    @pl.loop(0, nph)
    def _(p):
      base = base0 + p * ph
      pltpu.sync_copy(srcp_hbm.at[pl.ds(base, ph)], src_v)
      pltpu.sync_copy(dstp_hbm.at[pl.ds(base, ph)], dst_v)
      for j in range(nbuf):
        pltpu.async_copy(feat_hbm.at[src_v.at[j]], rows[j], sems[j])

      @pl.loop(0, ph // nbuf)
      def _(g):
        for j in range(nbuf):
          pltpu.make_async_copy(z128_hbm.at[pl.ds(0, ck)], rows[j],
                                sems[j]).wait()
          pltpu.sync_copy(rows[j], agg_sh.at[dst_v.at[nbuf * g + j]],
                          add=True)

          @pl.when(g < ph // nbuf - 1)
          def _():
            pltpu.async_copy(feat_hbm.at[src_v.at[nbuf * (g + 1) + j]],
                             rows[j], sems[j])

    plsc.subcore_barrier()

    @pl.loop(0, _NP // 16 // 128)
    def _(k):
      off = nbase + k * 128
      pltpu.sync_copy(agg_sh.at[pl.ds(off, 128)],
                      agg_hbm.at[pl.ds(c * _NP + off, 128)])

  return run(feat, srcp, dstp, z128)


def _sc_degree(dstp, z128, ones128):
  scratch = [
      pltpu.VMEM((_RPT, 128), jnp.int32),
      pltpu.VMEM((128, 128), jnp.float32),
      pltpu.VMEM_SHARED((_NP, 128), jnp.float32),
  ]

  @functools.partial(
      pl.kernel,
      out_type=jax.ShapeDtypeStruct((2 * _NP, 128), jnp.float32),
      mesh=plsc.VectorSubcoreMesh(**_MESH),
      scratch_types=scratch)
  def run(dstp_hbm, z128_hbm, ones128_hbm, deg_hbm,
          dst_v, ones_v, deg_sh):
    c = lax.axis_index("c")
    s = lax.axis_index("s")
    ebase = c * (_EROWS_P // 2) + s * _RPT
    nbase = s * (_NP // 16)

    pltpu.sync_copy(ones128_hbm, ones_v)

    @pl.loop(0, _NP // 16 // 128)
    def _(k):
      pltpu.sync_copy(z128_hbm, deg_sh.at[pl.ds(nbase + k * 128, 128)])

    pltpu.sync_copy(dstp_hbm.at[pl.ds(ebase, _RPT)], dst_v)
    plsc.subcore_barrier()

    @pl.loop(0, _RPT)
    def _(r):
      pltpu.sync_copy(ones_v, deg_sh.at[dst_v.at[r]], add=True)

    plsc.subcore_barrier()

    @pl.loop(0, _NP // 16 // 128)
    def _(k):
      off = nbase + k * 128
      pltpu.sync_copy(deg_sh.at[pl.ds(off, 128)],
                      deg_hbm.at[pl.ds(c * _NP + off, 128)])

  return run(dstp, z128, ones128)


def _dot_t(a, w):
  return lax.dot_general(a, w, (((1,), (1,)), ((), ())),
                         preferred_element_type=jnp.float32)


def _tc_layer_body(with_h, x_ref, a0_ref, a1_ref, d0_ref, d1_ref, b3_ref,
                   wrel_ref, wroot_ref, b_ref, *out_refs):
  if with_h:
    h_ref, pool_ref, cnt_ref = out_refs
  else:
    (pool_ref,) = out_refs
  agg = a0_ref[...] + a1_ref[...]
  deg = jnp.maximum((d0_ref[...] + d1_ref[...])[:, 0:1], 1.0)
  h = _dot_t(agg / deg, wrel_ref[...])
  h = h + _dot_t(x_ref[...], wroot_ref[...])
  h = jnp.maximum(h + b_ref[...], 0.0)
  if with_h:
    h_ref[...] = h
  brow = b3_ref[0]
  iot = lax.broadcasted_iota(jnp.int32, (_B, 1), 0)
  m = (brow == iot).astype(jnp.float32)
  psum = lax.dot_general(m, h, (((1,), (0,)), ((), ())),
                         preferred_element_type=jnp.float32)

  @pl.when(pl.program_id(0) == 0)
  def _():
    pool_ref[...] = jnp.zeros_like(pool_ref)
    if with_h:
      cnt_ref[...] = jnp.zeros_like(cnt_ref)

  pool_ref[...] += psum
  if with_h:
    ones8 = jnp.ones((_R, 8), jnp.float32)
    cnt_ref[...] += lax.dot_general(m, ones8, (((1,), (0,)), ((), ())),
                                    preferred_element_type=jnp.float32)


def _tc_layer(xp, agg, deg, batch3, w_rel, w_root, b, with_h):
  row = pl.BlockSpec((_R, 128), lambda i: (i, 0))
  in_specs = [
      row,
      pl.BlockSpec((_R, 128), lambda i: (i, 0)),
      pl.BlockSpec((_R, 128), lambda i: (i + _G, 0)),
      pl.BlockSpec((_R, 128), lambda i: (i, 0)),
      pl.BlockSpec((_R, 128), lambda i: (i + _G, 0)),
      pl.BlockSpec((1, 1, _R), lambda i: (i, 0, 0)),
      pl.BlockSpec((128, 128), lambda i: (0, 0)),
      pl.BlockSpec((128, 128), lambda i: (0, 0)),
      pl.BlockSpec((1, 128), lambda i: (0, 0)),
  ]
  out_shape = [jax.ShapeDtypeStruct((_B, 128), jnp.float32)]
  out_specs = [pl.BlockSpec((_B, 128), lambda i: (0, 0))]
  if with_h:
    out_shape = [jax.ShapeDtypeStruct((_NP, 128), jnp.float32)] + out_shape
    out_specs = [row] + out_specs
    out_shape.append(jax.ShapeDtypeStruct((_B, 8), jnp.float32))
    out_specs.append(pl.BlockSpec((_B, 8), lambda i: (0, 0)))
  return pl.pallas_call(
      functools.partial(_tc_layer_body, with_h),
      grid=(_G,),
      in_specs=in_specs,
      out_specs=out_specs,
      out_shape=out_shape,
  )(xp, agg, agg, deg, deg, batch3, w_rel, w_root, b.reshape(1, 128))


def _tc_head_body(p1_ref, p2_ref, cnt_ref, w1a_ref, w1b_ref, b1_ref,
                  w2_ref, b2_ref, out_ref):
  cnt = jnp.maximum(cnt_ref[...][:, 0:1], 1.0)
  a = _dot_t(p1_ref[...] / cnt, w1a_ref[...])
  a = a + _dot_t(p2_ref[...] / cnt, w1b_ref[...])
  a = jnp.maximum(a + b1_ref[...], 0.0)
  out_ref[...] = _dot_t(a, w2_ref[...]) + b2_ref[...]


def _tc_head(pool1, pool2, cnt, lin1_W, lin1_b, lin2_W, lin2_b):
  return pl.pallas_call(
      _tc_head_body,
      out_shape=jax.ShapeDtypeStruct((_B, 10), jnp.float32),
  )(pool1, pool2, cnt, lin1_W[:, :128], lin1_W[:, 128:],
    lin1_b.reshape(1, 128), lin2_W, lin2_b.reshape(1, 10))


def kernel(x, edge_index, batch, W1_rel, W1_root, b1, W2_rel, W2_root, b2,
           lin1_W, lin1_b, lin2_W, lin2_b):
  src = edge_index[0]
  dst = edge_index[1]
  pad = _EROWS_P * 128 - _E
  srcp = jnp.concatenate([src, jnp.zeros((pad,), jnp.int32)]).reshape(-1, 64)
  trash = _N + (jnp.arange(pad, dtype=jnp.int32) % (_NP - _N))
  dstflat = jnp.concatenate([dst, trash])
  dstp = dstflat.reshape(-1, 64)
  dstp128 = dstflat.reshape(-1, 128)
  xp = jnp.zeros((_NP, 128), jnp.float32).at[:_N].set(x)
  batch3 = jnp.concatenate([batch, jnp.full((_NP - _N,), _B, jnp.int32)])
  batch3 = batch3.reshape(_G, 1, _R)
  z128 = jnp.zeros((128, 128), jnp.float32)
  ones128 = jnp.ones((128, 128), jnp.float32)

  deg = _sc_degree(dstp128, z128, ones128)
  agg1 = _sc_aggregate(xp, srcp, dstp, z128)
  h1, pool1, cnt = _tc_layer(xp, agg1, deg, batch3, W1_rel, W1_root, b1, True)
  agg2 = _sc_aggregate(h1, srcp, dstp, z128)
  (pool2,) = _tc_layer(h1, agg2, deg, batch3, W2_rel, W2_root, b2, False)
  return _tc_head(pool1, pool2, cnt, lin1_W, lin1_b, lin2_W, lin2_b)
